# baseline (speedup 1.0000x reference)
import functools

import jax
import jax.numpy as jnp
from jax import lax
from jax.experimental import pallas as pl
from jax.experimental.pallas import tpu as pltpu
from jax.experimental.pallas import tpu_sc as plsc

N = 16384
C = 128
V = 64
CHUNK = 128


@functools.cache
def _make_sc_kernel():
    info = plsc.get_sparse_core_info()
    nc, ns, nl = info.num_cores, info.num_subcores, info.num_lanes
    nw = nc * ns
    b_per_w = N // nw
    n_chunks = b_per_w // CHUNK
    mesh = plsc.VectorSubcoreMesh(core_axis_name="c", subcore_axis_name="s")

    @functools.partial(
        pl.kernel,
        mesh=mesh,
        out_type=jax.ShapeDtypeStruct((N * C,), jnp.float32),
        compiler_params=pltpu.CompilerParams(needs_layout_passes=False),
        scratch_types=[
            pltpu.VMEM((b_per_w,), jnp.int32),
            pltpu.VMEM((C, 128), jnp.float32),
            pltpu.VMEM((V * C,), jnp.float32),
            pltpu.VMEM((C, CHUNK), jnp.float32),
            pltpu.VMEM((CHUNK * C,), jnp.float32),
        ],
    )
    def sc_k(inp1_hbm, idx_hbm, out_hbm, idx_v, traw_v, tbl_v, blk_v, out_v):
        wid = lax.axis_index("s") * nc + lax.axis_index("c")
        base = wid * b_per_w
        iota = jnp.arange(nl, dtype=jnp.int32)

        pltpu.sync_copy(idx_hbm.at[pl.ds(base, b_per_w)], idx_v)
        pltpu.sync_copy(inp1_hbm.at[:, pl.ds(0, 128)], traw_v)

        jvecs = [(j0 + iota) * C for j0 in range(0, V, nl)]

        def build_col(cc, _):
            ccf = jnp.full((nl,), cc, dtype=jnp.int32)
            for j0 in range(0, V, nl):
                v = traw_v[cc, pl.ds(j0, nl)]
                plsc.store_scatter(tbl_v, [jvecs[j0 // nl] + ccf], v * v)
            return _

        lax.fori_loop(0, C, build_col, None)

        iota128 = iota * C
        for k in range(n_chunks):
            col0 = base + k * CHUNK
            pltpu.sync_copy(inp1_hbm.at[:, pl.ds(col0, CHUNK)], blk_v)

            def group(g, _):
                idxv = idx_v[pl.ds(k * CHUNK + g * nl, nl)]
                idxv = jnp.minimum(jnp.maximum(idxv, 0), V - 1)
                pre = idxv * C
                spre = iota128 + g * (nl * C)

                @plsc.parallel_loop(0, C, unroll=8)
                def col(c):
                    t = blk_v[c, pl.ds(g * nl, nl)]
                    gv = plsc.load_gather(tbl_v, [pre + c])
                    plsc.store_scatter(out_v, [spre + c], t * gv)

                return _

            lax.fori_loop(0, CHUNK // nl, group, None)
            pltpu.sync_copy(out_v, out_hbm.at[pl.ds(col0 * C, CHUNK * C)])

    return sc_k


def kernel(inp1, inp2):
    idx = inp2.reshape(N).astype(jnp.int32)
    out = _make_sc_kernel()(inp1, idx)
    return (out.reshape(N, C),)


# speedup vs baseline: 1.3268x; 1.3268x over previous
import functools

import jax
import jax.numpy as jnp
from jax import lax
from jax.experimental import pallas as pl
from jax.experimental.pallas import tpu as pltpu
from jax.experimental.pallas import tpu_sc as plsc

N = 16384
C = 128
V = 64
TBL = 128
SUB = 128


def _table_body(inp1_ref, tbl_ref):
    x = inp1_ref[...]
    xt = jnp.transpose(x, (1, 0))
    tbl_ref[...] = xt * xt


def _build_table(inp1):
    return pl.pallas_call(
        _table_body,
        grid=(1,),
        in_specs=[pl.BlockSpec((C, TBL), lambda j: (0, 0))],
        out_specs=pl.BlockSpec((TBL, C), lambda j: (0, 0)),
        out_shape=jax.ShapeDtypeStruct((TBL, C), jnp.float32),
    )(inp1)


@functools.cache
def _make_sc_gather():
    info = plsc.get_sparse_core_info()
    nc, ns, nl = info.num_cores, info.num_subcores, info.num_lanes
    nw = nc * ns
    b_per_w = N // nw
    n_sub = b_per_w // SUB
    mesh = plsc.VectorSubcoreMesh(core_axis_name="c", subcore_axis_name="s")

    @functools.partial(
        pl.kernel,
        mesh=mesh,
        out_type=jax.ShapeDtypeStruct((N, C), jnp.float32),
        scratch_types=[
            pltpu.VMEM((b_per_w,), jnp.int32),
            [pltpu.VMEM((SUB, C), jnp.float32) for _ in range(n_sub)],
            [pltpu.SemaphoreType.DMA for _ in range(n_sub)],
            [pltpu.SemaphoreType.DMA for _ in range(n_sub)],
        ],
    )
    def gather_k(table_hbm, idx_hbm, out_hbm, idx_v, rows, gsems, ssems):
        wid = lax.axis_index("s") * nc + lax.axis_index("c")
        base = wid * b_per_w
        pltpu.sync_copy(idx_hbm.at[pl.ds(base, b_per_w)], idx_v)
        for i in range(b_per_w // nl):
            v = idx_v[pl.ds(i * nl, nl)]
            idx_v[pl.ds(i * nl, nl)] = jnp.minimum(jnp.maximum(v, 0), V - 1)
        gs = [
            pltpu.async_copy(
                table_hbm.at[idx_v.at[pl.ds(k * SUB, SUB)]], rows[k], gsems[k]
            )
            for k in range(n_sub)
        ]
        ss = []
        for k in range(n_sub):
            gs[k].wait()
            ss.append(
                pltpu.async_copy(
                    rows[k], out_hbm.at[pl.ds(base + k * SUB, SUB)], ssems[k]
                )
            )
        for s in ss:
            s.wait()

    return gather_k


_BLK = 2048


def _mul_body(inp1_ref, g_ref, o_ref):
    o_ref[...] = jnp.transpose(inp1_ref[...], (1, 0)) * g_ref[...]


def _mul(inp1, g):
    return pl.pallas_call(
        _mul_body,
        grid=(N // _BLK,),
        in_specs=[
            pl.BlockSpec((C, _BLK), lambda j: (0, j)),
            pl.BlockSpec((_BLK, C), lambda j: (j, 0)),
        ],
        out_specs=pl.BlockSpec((_BLK, C), lambda j: (j, 0)),
        out_shape=jax.ShapeDtypeStruct((N, C), jnp.float32),
    )(inp1, g)


def kernel(inp1, inp2):
    idx = inp2.reshape(N).astype(jnp.int32)
    table = _build_table(inp1)
    g = _make_sc_gather()(table, idx)
    out = _mul(inp1, g)
    return (out,)
